# baseline (speedup 1.0000x reference)
import functools

import jax
import jax.numpy as jnp
from jax import lax
from jax.experimental import pallas as pl
from jax.experimental.pallas import tpu as pltpu
from jax.experimental.pallas import tpu_sc as plsc

SIZE = 100000
DIM = 128
B = 4
NNODES = 1024
TOPN = 50

GRID1 = 32
BLK1 = 3200
PAD = GRID1 * BLK1

NC, NS, NL = 2, 16, 16
TPB = (NC * NS) // B
CHUNK = PAD // TPB
NVR = CHUNK // NL
TCAP = 64
MCAP = TPB * TCAP
MVR = MCAP // NL

_BIG = 2**30


def _dist_body(nf_ref, means_ref, stds_ref, ds_ref, mean_ref, std_ref):
    i = pl.program_id(0)

    @pl.when(i == 0)
    def _stats():
        nf = nf_ref[...]
        mean = jnp.mean(nf, axis=1)
        xc = nf - mean[:, None, :]
        mean_ref[...] = mean
        std_ref[...] = jnp.sqrt(jnp.mean(xc * xc, axis=1))

    mb = means_ref[...]
    sb = stds_ref[...]
    mu = mean_ref[...]
    sg = std_ref[...]

    dn = (((1,), (1,)), ((), ()))
    ones = jnp.ones((1, DIM), jnp.float32)
    cross_m = lax.dot_general(mu, mb, dn, preferred_element_type=jnp.float32)
    cross_s = lax.dot_general(sg, sb, dn, preferred_element_type=jnp.float32)
    m2 = lax.dot_general(ones, mb * mb, dn, preferred_element_type=jnp.float32)
    s2 = lax.dot_general(ones, sb * sb, dn, preferred_element_type=jnp.float32)
    mu2 = jnp.sum(mu * mu, axis=1)[:, None]
    sg2 = jnp.sum(sg * sg, axis=1)[:, None]

    am = jnp.maximum(m2 - 2.0 * cross_m + mu2, 0.0)
    asd = jnp.maximum(s2 - 2.0 * cross_s + sg2, 0.0)
    ds = jnp.sqrt(am) + jnp.sqrt(asd)

    col = i * BLK1 + lax.broadcasted_iota(jnp.int32, (B, BLK1), 1)
    ds_ref[...] = jnp.where(col < SIZE, ds, jnp.inf)


def _sc_body(ds_hbm, means_hbm, stds_hbm, temp_hbm, mg_hbm, sg_hbm,
             dsv, candv, candi, topv, topi, mflatv, mflati, wbuf,
             rows_m, rows_s, stage_m, stage_s, tempv,
             stv_s, sti_s, thr_s, sem):
    c = lax.axis_index("c")
    s = lax.axis_index("s")
    b_loc = s // TPB
    j = s % TPB
    b = c * 2 + b_loc

    inf = jnp.float32(jnp.inf)
    full_inf = jnp.full((NL,), inf, jnp.float32)
    zeros_i = jnp.zeros((NL,), jnp.int32)
    lanes = lax.broadcasted_iota(jnp.int32, (NL,), 0)
    lane0 = lanes == 0

    pltpu.sync_copy(ds_hbm.at[b, pl.ds(j * CHUNK, CHUNK)], dsv)

    def p1(i, ms):
        m1, m2, m3, m4 = ms
        v = dsv[pl.ds(i * NL, NL)]
        h1 = jnp.maximum(m1, v)
        m1 = jnp.minimum(m1, v)
        h2 = jnp.maximum(m2, h1)
        m2 = jnp.minimum(m2, h1)
        h3 = jnp.maximum(m3, h2)
        m3 = jnp.minimum(m3, h2)
        m4 = jnp.minimum(m4, h3)
        return (m1, m2, m3, m4)

    _, _, _, m4 = lax.fori_loop(0, NVR, p1, (full_inf,) * 4)
    tloc = jnp.max(m4)

    stage_m[pl.ds(0, NL)] = jnp.full((NL,), tloc, jnp.float32)
    pltpu.sync_copy(stage_m.at[pl.ds(0, NL)], thr_s.at[b_loc, pl.ds(j * NL, NL)])
    plsc.subcore_barrier()
    pltpu.sync_copy(thr_s.at[b_loc], stage_s)
    tminv = full_inf
    for u in range(TPB):
        tminv = jnp.minimum(tminv, stage_s[pl.ds(u * NL, NL)])
    tsp = jnp.full((NL,), jnp.min(tminv), jnp.float32)

    gbase = j * CHUNK

    def comp(i, off):
        vs = [dsv[pl.ds(i * (4 * NL) + u * NL, NL)] for u in range(4)]
        ms = [v <= tsp for v in vs]
        any4 = (ms[0] | ms[1]) | (ms[2] | ms[3])
        n4 = jnp.max(plsc.all_reduce_population_count(any4))
        cs = [jnp.max(plsc.all_reduce_population_count(m)) for m in ms]

        @pl.when(n4 > 0)
        def _do():
            o = off
            for u in range(4):
                gi = jnp.full((NL,), gbase + i * (4 * NL) + u * NL, jnp.int32) + lanes
                plsc.store_compressed(candv.at[pl.ds(o, NL)], vs[u], mask=ms[u])
                plsc.store_compressed(candi.at[pl.ds(o, NL)], gi, mask=ms[u])
                o = o + cs[u]

        return off + cs[0] + cs[1] + cs[2] + cs[3]

    off = lax.fori_loop(0, NVR // 4, comp, jnp.int32(0))
    candv[pl.ds(off, NL)] = full_inf
    candi[pl.ds(off, NL)] = zeros_i
    ncv = (off + NL) // NL

    def _select_topk(valref, idxref, nv, outv, outi):
        def sel(k, carry):
            def mloop(i, acc):
                return jnp.minimum(acc, valref[pl.ds(i * NL, NL)])
            m = jnp.min(lax.fori_loop(0, nv, mloop, full_inf))
            msp = jnp.full((NL,), m, jnp.float32)

            def floop(i, pos):
                eq = valref[pl.ds(i * NL, NL)] == msp
                cnt = jnp.max(plsc.all_reduce_population_count(eq))
                f = jnp.max(plsc.all_reduce_ffs(eq))
                return jnp.minimum(pos, jnp.where(cnt > 0, i * NL + f, _BIG))

            pos = lax.fori_loop(0, nv, floop, jnp.int32(_BIG))
            possp = jnp.full((NL,), pos, jnp.int32)
            gi = plsc.load_gather(idxref, [possp])
            plsc.store_scatter(valref, [possp], full_inf, mask=lane0)
            ksp = jnp.full((NL,), k, jnp.int32)
            plsc.store_scatter(outv, [ksp], msp, mask=lane0)
            plsc.store_scatter(outi, [ksp], gi, mask=lane0)
            return carry

        for q in range(TCAP // NL):
            outv[pl.ds(q * NL, NL)] = full_inf
            outi[pl.ds(q * NL, NL)] = q * NL + lanes
        lax.fori_loop(0, TOPN, sel, jnp.int32(0))

    _select_topk(candv, candi, ncv, topv, topi)

    pltpu.sync_copy(topv, stv_s.at[b_loc, pl.ds(j * TCAP, TCAP)])
    pltpu.sync_copy(topi, sti_s.at[b_loc, pl.ds(j * TCAP, TCAP)])
    plsc.subcore_barrier()

    @pl.when(j == 0)
    def _merge():
        pltpu.sync_copy(stv_s.at[b_loc], mflatv)
        pltpu.sync_copy(sti_s.at[b_loc], mflati)

        l8 = lanes < TPB
        lanebase = jnp.where(l8, lanes * TCAP, 0)

        def mstep(k, ptr):
            idxv = lanebase + ptr
            hv = jnp.where(l8, plsc.load_gather(mflatv, [idxv]), inf)
            gv = plsc.load_gather(mflati, [idxv])
            m = jnp.min(hv)
            eq = (hv == jnp.full((NL,), m, jnp.float32)) & l8
            f = jnp.max(plsc.all_reduce_ffs(eq))
            win = lanes == f
            ksp = jnp.full((NL,), k, jnp.int32)
            plsc.store_scatter(topv, [ksp], hv, mask=win)
            plsc.store_scatter(topi, [ksp], gv, mask=win)
            return jnp.where(win, ptr + 1, ptr)

        lax.fori_loop(0, TOPN, mstep, jnp.zeros((NL,), jnp.int32))

        pltpu.sync_copy(temp_hbm, tempv)
        tv = tempv[...]
        svs = []
        for q in range(TCAP // NL):
            slot = q * NL + lanes
            vq = topv[pl.ds(q * NL, NL)]
            sq = jnp.where(slot < TOPN, jnp.exp(-tv * vq), -inf)
            svs.append(sq)
        msx = jnp.max(jnp.maximum(jnp.maximum(svs[0], svs[1]),
                                  jnp.maximum(svs[2], svs[3])))
        eqs, tot = [], jnp.float32(0.0)
        for q in range(TCAP // NL):
            slot = q * NL + lanes
            eq = jnp.where(slot < TOPN, jnp.exp(svs[q] - msx), 0.0)
            eqs.append(eq)
            tot = tot + jnp.sum(eq)
        for q in range(TCAP // NL):
            wbuf[pl.ds(q * NL, NL)] = eqs[q] / tot

        gm = pltpu.async_copy(means_hbm.at[topi], rows_m, sem)
        gs = pltpu.async_copy(stds_hbm.at[topi], rows_s, sem)
        gm.wait()
        gs.wait()

        def comb(k, accs):
            ksp = jnp.full((NL,), k, jnp.int32)
            wk = plsc.load_gather(wbuf, [ksp])
            out = []
            for ci in range(DIM // NL):
                out.append(accs[ci] + wk * rows_m[k, pl.ds(ci * NL, NL)])
            for ci in range(DIM // NL):
                out.append(accs[DIM // NL + ci] + wk * rows_s[k, pl.ds(ci * NL, NL)])
            return tuple(out)

        zero = jnp.zeros((NL,), jnp.float32)
        accs = lax.fori_loop(0, TCAP, comb, (zero,) * (2 * (DIM // NL)))
        for ci in range(DIM // NL):
            stage_m[pl.ds(ci * NL, NL)] = accs[ci]
            stage_s[pl.ds(ci * NL, NL)] = accs[DIM // NL + ci]
        pltpu.sync_copy(stage_m, mg_hbm.at[b])
        pltpu.sync_copy(stage_s, sg_hbm.at[b])


def _final_body(nf_ref, mean_ref, std_ref, mg_ref, sg_ref, fl_ref, out_ref):
    lf = 1.0 / (1.0 + jnp.exp(-fl_ref[0, 0]))
    mean = mean_ref[...]
    std = std_ref[...]
    mean_final = lf * mg_ref[...] + (1.0 - lf) * mean
    std_final = lf * sg_ref[...] + (1.0 - lf) * std
    nf = nf_ref[...]
    out_ref[...] = (std_final[:, None, :] * (nf - mean[:, None, :]) / std[:, None, :]
                    + mean_final[:, None, :])


def kernel(node_fts, means, stds, temp, fixed_lerp):
    f32 = jnp.float32

    ds, mean, std = pl.pallas_call(
        _dist_body,
        grid=(GRID1,),
        in_specs=[
            pl.BlockSpec((B, NNODES, DIM), lambda i: (0, 0, 0)),
            pl.BlockSpec((BLK1, DIM), lambda i: (i, 0)),
            pl.BlockSpec((BLK1, DIM), lambda i: (i, 0)),
        ],
        out_specs=(
            pl.BlockSpec((B, BLK1), lambda i: (0, i)),
            pl.BlockSpec((B, DIM), lambda i: (0, 0)),
            pl.BlockSpec((B, DIM), lambda i: (0, 0)),
        ),
        out_shape=(jax.ShapeDtypeStruct((B, PAD), f32),
                   jax.ShapeDtypeStruct((B, DIM), f32),
                   jax.ShapeDtypeStruct((B, DIM), f32)),
    )(node_fts, means, stds)

    sc_mid = pl.kernel(
        _sc_body,
        out_type=(jax.ShapeDtypeStruct((B, DIM), f32),
                  jax.ShapeDtypeStruct((B, DIM), f32)),
        mesh=plsc.VectorSubcoreMesh(core_axis_name="c", subcore_axis_name="s",
                                    num_cores=NC, num_subcores=NS),
        compiler_params=pltpu.CompilerParams(needs_layout_passes=False),
        scratch_types=[
            pltpu.VMEM((CHUNK,), f32),
            pltpu.VMEM((CHUNK + NL,), f32),
            pltpu.VMEM((CHUNK + NL,), jnp.int32),
            pltpu.VMEM((TCAP,), f32),
            pltpu.VMEM((TCAP,), jnp.int32),
            pltpu.VMEM((MCAP,), f32),
            pltpu.VMEM((MCAP,), jnp.int32),
            pltpu.VMEM((TCAP,), f32),
            pltpu.VMEM((TCAP, DIM), f32),
            pltpu.VMEM((TCAP, DIM), f32),
            pltpu.VMEM((DIM,), f32),
            pltpu.VMEM((DIM,), f32),
            pltpu.VMEM((NL,), f32),
            pltpu.VMEM_SHARED((2, MCAP), f32),
            pltpu.VMEM_SHARED((2, MCAP), jnp.int32),
            pltpu.VMEM_SHARED((2, DIM), f32),
            pltpu.SemaphoreType.DMA,
        ],
    )
    temp16 = jnp.full((NL,), temp, f32)
    mg, sg = sc_mid(ds, means, stds, temp16)

    out = pl.pallas_call(
        _final_body,
        out_shape=jax.ShapeDtypeStruct((B, NNODES, DIM), f32),
    )(node_fts, mean, std, mg, sg, fixed_lerp.reshape(1, 1))
    return out


# speedup vs baseline: 1.1601x; 1.1601x over previous
import functools

import jax
import jax.numpy as jnp
from jax import lax
from jax.experimental import pallas as pl
from jax.experimental.pallas import tpu as pltpu
from jax.experimental.pallas import tpu_sc as plsc

SIZE = 100000
DIM = 128
B = 4
NNODES = 1024
TOPN = 50

GRID1 = 8
BLK1 = 12800
PAD = GRID1 * BLK1

NC, NS, NL = 2, 16, 16
TPB = (NC * NS) // B
CHUNK = PAD // TPB
NVR = CHUNK // NL
TCAP = 64
MCAP = TPB * TCAP
MVR = MCAP // NL

_BIG = 2**30


def _dist_body(nf_ref, means_ref, stds_ref, ds_ref, mean_ref, std_ref):
    i = pl.program_id(0)

    @pl.when(i == 0)
    def _stats():
        nf = nf_ref[...]
        mean = jnp.mean(nf, axis=1)
        xc = nf - mean[:, None, :]
        mean_ref[...] = mean
        std_ref[...] = jnp.sqrt(jnp.mean(xc * xc, axis=1))

    mb = means_ref[...]
    sb = stds_ref[...]
    mu = mean_ref[...]
    sg = std_ref[...]

    dn = (((1,), (1,)), ((), ()))
    ones = jnp.ones((1, DIM), jnp.float32)
    cross_m = lax.dot_general(mu, mb, dn, preferred_element_type=jnp.float32)
    cross_s = lax.dot_general(sg, sb, dn, preferred_element_type=jnp.float32)
    m2 = lax.dot_general(ones, mb * mb, dn, preferred_element_type=jnp.float32)
    s2 = lax.dot_general(ones, sb * sb, dn, preferred_element_type=jnp.float32)
    mu2 = jnp.sum(mu * mu, axis=1)[:, None]
    sg2 = jnp.sum(sg * sg, axis=1)[:, None]

    am = jnp.maximum(m2 - 2.0 * cross_m + mu2, 0.0)
    asd = jnp.maximum(s2 - 2.0 * cross_s + sg2, 0.0)
    ds = jnp.sqrt(am) + jnp.sqrt(asd)

    col = i * BLK1 + lax.broadcasted_iota(jnp.int32, (B, BLK1), 1)
    ds_ref[...] = jnp.where(col < SIZE, ds, jnp.inf)


def _sc_body(ds_hbm, means_hbm, stds_hbm, temp_hbm, mg_hbm, sg_hbm,
             dsv, candv, candi, topv, topi, mflatv, mflati, wbuf,
             rows_m, rows_s, stage_m, stage_s, tempv,
             stv_s, sti_s, thr_s, sem):
    c = lax.axis_index("c")
    s = lax.axis_index("s")
    b_loc = s // TPB
    j = s % TPB
    b = c * 2 + b_loc

    inf = jnp.float32(jnp.inf)
    full_inf = jnp.full((NL,), inf, jnp.float32)
    zeros_i = jnp.zeros((NL,), jnp.int32)
    lanes = lax.broadcasted_iota(jnp.int32, (NL,), 0)
    lane0 = lanes == 0

    pltpu.sync_copy(ds_hbm.at[b, pl.ds(j * CHUNK, CHUNK)], dsv)

    def p1(i, ms):
        m1, m2, m3, m4 = ms
        v = dsv[pl.ds(i * NL, NL)]
        h1 = jnp.maximum(m1, v)
        m1 = jnp.minimum(m1, v)
        h2 = jnp.maximum(m2, h1)
        m2 = jnp.minimum(m2, h1)
        h3 = jnp.maximum(m3, h2)
        m3 = jnp.minimum(m3, h2)
        m4 = jnp.minimum(m4, h3)
        return (m1, m2, m3, m4)

    _, _, _, m4 = lax.fori_loop(0, NVR, p1, (full_inf,) * 4)
    tloc = jnp.max(m4)

    stage_m[pl.ds(0, NL)] = jnp.full((NL,), tloc, jnp.float32)
    pltpu.sync_copy(stage_m.at[pl.ds(0, NL)], thr_s.at[b_loc, pl.ds(j * NL, NL)])
    plsc.subcore_barrier()
    pltpu.sync_copy(thr_s.at[b_loc], stage_s)
    tminv = full_inf
    for u in range(TPB):
        tminv = jnp.minimum(tminv, stage_s[pl.ds(u * NL, NL)])
    tsp = jnp.full((NL,), jnp.min(tminv), jnp.float32)

    gbase = j * CHUNK

    def comp(i, off):
        vs = [dsv[pl.ds(i * (4 * NL) + u * NL, NL)] for u in range(4)]
        ms = [v <= tsp for v in vs]
        any4 = (ms[0] | ms[1]) | (ms[2] | ms[3])
        n4 = jnp.max(plsc.all_reduce_population_count(any4))
        cs = [jnp.max(plsc.all_reduce_population_count(m)) for m in ms]

        @pl.when(n4 > 0)
        def _do():
            o = off
            for u in range(4):
                gi = jnp.full((NL,), gbase + i * (4 * NL) + u * NL, jnp.int32) + lanes
                plsc.store_compressed(candv.at[pl.ds(o, NL)], vs[u], mask=ms[u])
                plsc.store_compressed(candi.at[pl.ds(o, NL)], gi, mask=ms[u])
                o = o + cs[u]

        return off + cs[0] + cs[1] + cs[2] + cs[3]

    off = lax.fori_loop(0, NVR // 4, comp, jnp.int32(0))
    candv[pl.ds(off, NL)] = full_inf
    candi[pl.ds(off, NL)] = zeros_i
    ncv = (off + NL) // NL

    def _select_topk(valref, idxref, nv, outv, outi):
        def sel(k, carry):
            def mloop(i, acc):
                return jnp.minimum(acc, valref[pl.ds(i * NL, NL)])
            m = jnp.min(lax.fori_loop(0, nv, mloop, full_inf))
            msp = jnp.full((NL,), m, jnp.float32)

            def floop(i, pos):
                eq = valref[pl.ds(i * NL, NL)] == msp
                cnt = jnp.max(plsc.all_reduce_population_count(eq))
                f = jnp.max(plsc.all_reduce_ffs(eq))
                return jnp.minimum(pos, jnp.where(cnt > 0, i * NL + f, _BIG))

            pos = lax.fori_loop(0, nv, floop, jnp.int32(_BIG))
            possp = jnp.full((NL,), pos, jnp.int32)
            gi = plsc.load_gather(idxref, [possp])
            plsc.store_scatter(valref, [possp], full_inf, mask=lane0)
            ksp = jnp.full((NL,), k, jnp.int32)
            plsc.store_scatter(outv, [ksp], msp, mask=lane0)
            plsc.store_scatter(outi, [ksp], gi, mask=lane0)
            return carry

        for q in range(TCAP // NL):
            outv[pl.ds(q * NL, NL)] = full_inf
            outi[pl.ds(q * NL, NL)] = q * NL + lanes
        lax.fori_loop(0, TOPN, sel, jnp.int32(0))

    _select_topk(candv, candi, ncv, topv, topi)

    pltpu.sync_copy(topv, stv_s.at[b_loc, pl.ds(j * TCAP, TCAP)])
    pltpu.sync_copy(topi, sti_s.at[b_loc, pl.ds(j * TCAP, TCAP)])
    plsc.subcore_barrier()

    @pl.when(j == 0)
    def _merge():
        pltpu.sync_copy(stv_s.at[b_loc], mflatv)
        pltpu.sync_copy(sti_s.at[b_loc], mflati)

        l8 = lanes < TPB
        lanebase = jnp.where(l8, lanes * TCAP, 0)

        def mstep(k, ptr):
            idxv = lanebase + ptr
            hv = jnp.where(l8, plsc.load_gather(mflatv, [idxv]), inf)
            gv = plsc.load_gather(mflati, [idxv])
            m = jnp.min(hv)
            eq = (hv == jnp.full((NL,), m, jnp.float32)) & l8
            f = jnp.max(plsc.all_reduce_ffs(eq))
            win = lanes == f
            ksp = jnp.full((NL,), k, jnp.int32)
            plsc.store_scatter(topv, [ksp], hv, mask=win)
            plsc.store_scatter(topi, [ksp], gv, mask=win)
            return jnp.where(win, ptr + 1, ptr)

        lax.fori_loop(0, TOPN, mstep, jnp.zeros((NL,), jnp.int32))

        pltpu.sync_copy(temp_hbm, tempv)
        tv = tempv[...]
        svs = []
        for q in range(TCAP // NL):
            slot = q * NL + lanes
            vq = topv[pl.ds(q * NL, NL)]
            sq = jnp.where(slot < TOPN, jnp.exp(-tv * vq), -inf)
            svs.append(sq)
        msx = jnp.max(jnp.maximum(jnp.maximum(svs[0], svs[1]),
                                  jnp.maximum(svs[2], svs[3])))
        eqs, tot = [], jnp.float32(0.0)
        for q in range(TCAP // NL):
            slot = q * NL + lanes
            eq = jnp.where(slot < TOPN, jnp.exp(svs[q] - msx), 0.0)
            eqs.append(eq)
            tot = tot + jnp.sum(eq)
        for q in range(TCAP // NL):
            wbuf[pl.ds(q * NL, NL)] = eqs[q] / tot

        gm = pltpu.async_copy(means_hbm.at[topi], rows_m, sem)
        gs = pltpu.async_copy(stds_hbm.at[topi], rows_s, sem)
        gm.wait()
        gs.wait()

        def comb(k, accs):
            ksp = jnp.full((NL,), k, jnp.int32)
            wk = plsc.load_gather(wbuf, [ksp])
            out = []
            for ci in range(DIM // NL):
                out.append(accs[ci] + wk * rows_m[k, pl.ds(ci * NL, NL)])
            for ci in range(DIM // NL):
                out.append(accs[DIM // NL + ci] + wk * rows_s[k, pl.ds(ci * NL, NL)])
            return tuple(out)

        zero = jnp.zeros((NL,), jnp.float32)
        accs = lax.fori_loop(0, TCAP, comb, (zero,) * (2 * (DIM // NL)))
        for ci in range(DIM // NL):
            stage_m[pl.ds(ci * NL, NL)] = accs[ci]
            stage_s[pl.ds(ci * NL, NL)] = accs[DIM // NL + ci]
        pltpu.sync_copy(stage_m, mg_hbm.at[b])
        pltpu.sync_copy(stage_s, sg_hbm.at[b])


def _final_body(nf_ref, mean_ref, std_ref, mg_ref, sg_ref, fl_ref, out_ref):
    lf = 1.0 / (1.0 + jnp.exp(-fl_ref[0, 0]))
    mean = mean_ref[...]
    std = std_ref[...]
    mean_final = lf * mg_ref[...] + (1.0 - lf) * mean
    std_final = lf * sg_ref[...] + (1.0 - lf) * std
    nf = nf_ref[...]
    out_ref[...] = (std_final[:, None, :] * (nf - mean[:, None, :]) / std[:, None, :]
                    + mean_final[:, None, :])


def kernel(node_fts, means, stds, temp, fixed_lerp):
    f32 = jnp.float32

    ds, mean, std = pl.pallas_call(
        _dist_body,
        grid=(GRID1,),
        in_specs=[
            pl.BlockSpec((B, NNODES, DIM), lambda i: (0, 0, 0)),
            pl.BlockSpec((BLK1, DIM), lambda i: (i, 0)),
            pl.BlockSpec((BLK1, DIM), lambda i: (i, 0)),
        ],
        out_specs=(
            pl.BlockSpec((B, BLK1), lambda i: (0, i)),
            pl.BlockSpec((B, DIM), lambda i: (0, 0)),
            pl.BlockSpec((B, DIM), lambda i: (0, 0)),
        ),
        out_shape=(jax.ShapeDtypeStruct((B, PAD), f32),
                   jax.ShapeDtypeStruct((B, DIM), f32),
                   jax.ShapeDtypeStruct((B, DIM), f32)),
    )(node_fts, means, stds)

    sc_mid = pl.kernel(
        _sc_body,
        out_type=(jax.ShapeDtypeStruct((B, DIM), f32),
                  jax.ShapeDtypeStruct((B, DIM), f32)),
        mesh=plsc.VectorSubcoreMesh(core_axis_name="c", subcore_axis_name="s",
                                    num_cores=NC, num_subcores=NS),
        compiler_params=pltpu.CompilerParams(needs_layout_passes=False),
        scratch_types=[
            pltpu.VMEM((CHUNK,), f32),
            pltpu.VMEM((CHUNK + NL,), f32),
            pltpu.VMEM((CHUNK + NL,), jnp.int32),
            pltpu.VMEM((TCAP,), f32),
            pltpu.VMEM((TCAP,), jnp.int32),
            pltpu.VMEM((MCAP,), f32),
            pltpu.VMEM((MCAP,), jnp.int32),
            pltpu.VMEM((TCAP,), f32),
            pltpu.VMEM((TCAP, DIM), f32),
            pltpu.VMEM((TCAP, DIM), f32),
            pltpu.VMEM((DIM,), f32),
            pltpu.VMEM((DIM,), f32),
            pltpu.VMEM((NL,), f32),
            pltpu.VMEM_SHARED((2, MCAP), f32),
            pltpu.VMEM_SHARED((2, MCAP), jnp.int32),
            pltpu.VMEM_SHARED((2, DIM), f32),
            pltpu.SemaphoreType.DMA,
        ],
    )
    temp16 = jnp.full((NL,), temp, f32)
    mg, sg = sc_mid(ds, means, stds, temp16)

    out = pl.pallas_call(
        _final_body,
        out_shape=jax.ShapeDtypeStruct((B, NNODES, DIM), f32),
    )(node_fts, mean, std, mg, sg, fixed_lerp.reshape(1, 1))
    return out
